# baseline (speedup 1.0000x reference)
import functools

import jax
import jax.numpy as jnp
from jax.experimental import pallas as pl
from jax.experimental.pallas import tpu as pltpu

_CONF_T = 0.25
_IOU_T = 0.45
_MAX_WH = 4096.0
_MAX_DET = 300
_OUT_R = 304
_KS, _KL = 8, 48


def _prep_body(pred_ref, out_ref):
    pred = pred_ref[...]
    xywh = pred[:, 0:4]
    obj = pred[:, 4:5]
    cls = pred[:, 5:]
    scores = cls * obj
    conf = jnp.max(scores, axis=1, keepdims=True)
    lane = jax.lax.broadcasted_iota(jnp.int32, scores.shape, 1).astype(
        jnp.float32)
    j = jnp.min(jnp.where(scores >= conf, lane, 3.0e8), axis=1, keepdims=True)
    x = xywh[:, 0:1]
    y = xywh[:, 1:2]
    w = xywh[:, 2:3]
    h = xywh[:, 3:4]
    x1 = x - w / 2.0
    y1 = y - h / 2.0
    x2 = x + w / 2.0
    y2 = y + h / 2.0
    valid = (obj > _CONF_T) & (conf > _CONF_T)
    sw = jnp.where(valid, conf, -1.0)
    zero = jnp.zeros_like(sw)
    out_ref[...] = jnp.concatenate([x1, y1, x2, y2, conf, j, sw, zero], axis=1)


def _nms_body(n_boxes, data_ref, order_ref, out_ref,
              kx1_ref, ky1_ref, kx2_ref, ky2_ref, ka_ref):
    out_ref[...] = jnp.zeros((_OUT_R, 8), jnp.float32)
    sentinel = jnp.full((_KS, _KL), -1.0e9, jnp.float32)
    kx1_ref[...] = sentinel
    ky1_ref[...] = sentinel
    kx2_ref[...] = sentinel
    ky2_ref[...] = sentinel
    ka_ref[...] = sentinel
    ids = (jax.lax.broadcasted_iota(jnp.int32, (_KS, _KL), 1) * _KS
           + jax.lax.broadcasted_iota(jnp.int32, (_KS, _KL), 0))
    lane8 = jax.lax.broadcasted_iota(jnp.int32, (1, 8), 1)

    def load(i):
        ii = jnp.minimum(i, n_boxes - 1)
        pos = jnp.sum(order_ref[pl.ds(ii, 1), :])
        row = data_ref[pl.ds(pos, 1), :]
        score = jnp.sum(jnp.where(lane8 == 6, row, 0.0))
        return pos, row, score

    st0 = load(jnp.int32(0))

    def cond(st):
        i, cnt, _, _, score = st
        return (i < n_boxes) & (cnt < _MAX_DET) & (score > 0.0)

    def body(st):
        i, cnt, pos, row, _ = st

        def bc(k):
            return jnp.broadcast_to(row[0:1, k:k + 1], (_KS, _KL))

        off = bc(5) * _MAX_WH
        x1o = bc(0) + off
        y1o = bc(1) + off
        x2o = bc(2) + off
        y2o = bc(3) + off
        carea = (x2o - x1o) * (y2o - y1o)
        kx1 = kx1_ref[...]
        ky1 = ky1_ref[...]
        kx2 = kx2_ref[...]
        ky2 = ky2_ref[...]
        ka = ka_ref[...]
        ltx = jnp.maximum(kx1, x1o)
        lty = jnp.maximum(ky1, y1o)
        rbx = jnp.minimum(kx2, x2o)
        rby = jnp.minimum(ky2, y2o)
        inter = jnp.maximum(rbx - ltx, 0.0) * jnp.maximum(rby - lty, 0.0)
        iou = inter / (ka + carea - inter + 1e-9)
        hit = (ids < cnt) & (iou > _IOU_T)
        keep = jnp.max(jnp.where(hit, 1.0, 0.0)) == 0.0
        m = (ids == cnt) & keep
        kx1_ref[...] = jnp.where(m, x1o, kx1)
        ky1_ref[...] = jnp.where(m, y1o, ky1)
        kx2_ref[...] = jnp.where(m, x2o, kx2)
        ky2_ref[...] = jnp.where(m, y2o, ky2)
        ka_ref[...] = jnp.where(m, carea, ka)
        outrow = jnp.where(lane8 == 7, pos.astype(jnp.float32), row)
        out_ref[pl.ds(cnt, 1), :] = jnp.where(keep, outrow, 0.0)
        cnt2 = cnt + jnp.where(keep, 1, 0).astype(jnp.int32)
        pos2, row2, score2 = load(i + 1)
        return i + 1, cnt2, pos2, row2, score2

    jax.lax.while_loop(cond, body, (jnp.int32(0), jnp.int32(0)) + st0)


def kernel(prediction, logits):
    b, n, nf = prediction.shape
    f32 = jnp.float32

    tp = 2500 if n % 2500 == 0 else n
    nt = n // tp
    pred4 = prediction.reshape(b, nt, tp, nf)

    data = pl.pallas_call(
        _prep_body,
        grid=(b, nt),
        in_specs=[
            pl.BlockSpec((None, None, tp, nf), lambda bb, t: (bb, t, 0, 0)),
        ],
        out_specs=pl.BlockSpec((None, None, tp, 8),
                               lambda bb, t: (bb, t, 0, 0)),
        out_shape=jax.ShapeDtypeStruct((b, nt, tp, 8), f32),
    )(pred4).reshape(b, n, 8)

    sw = data[..., 6]
    iot = jax.lax.broadcasted_iota(jnp.int32, (b, n), 1)
    _, order = jax.lax.sort((-sw, iot), dimension=1, is_stable=True,
                            num_keys=1)
    order3 = order[..., None]

    dets_full = pl.pallas_call(
        functools.partial(_nms_body, n),
        grid=(b,),
        in_specs=[
            pl.BlockSpec((None, n, 8), lambda bb: (bb, 0, 0)),
            pl.BlockSpec((None, n, 1), lambda bb: (bb, 0, 0)),
        ],
        out_specs=pl.BlockSpec((None, _OUT_R, 8), lambda bb: (bb, 0, 0)),
        out_shape=jax.ShapeDtypeStruct((b, _OUT_R, 8), f32),
        scratch_shapes=[pltpu.VMEM((_KS, _KL), f32) for _ in range(5)],
    )(data, order3)

    dets = dets_full[:, :_MAX_DET, :6]
    idx = dets_full[:, :_MAX_DET, 7].astype(jnp.int32)
    validm = dets_full[:, :_MAX_DET, 4:5] > 0.0
    logs = jnp.take_along_axis(logits, idx[..., None], axis=1)
    logs = jnp.where(validm, logs, 0.0)
    return dets, logs


# speedup vs baseline: 1.0539x; 1.0539x over previous
import functools

import jax
import jax.numpy as jnp
from jax.experimental import pallas as pl
from jax.experimental.pallas import tpu as pltpu

_CONF_T = 0.25
_IOU_T = 0.45
_MAX_WH = 4096.0
_MAX_DET = 300
_OUT_R = 304
_KS, _KL = 8, 48
_DW = 16


def _prep_body(pred_ref, out_ref):
    n = pred_ref.shape[0]
    chunk = 2000 if n % 2000 == 0 else n
    for c in range(0, n, chunk):
        pred = pred_ref[c:c + chunk, :]
        xywh = pred[:, 0:4]
        obj = pred[:, 4:5]
        cls = pred[:, 5:]
        scores = cls * obj
        conf = jnp.max(scores, axis=1, keepdims=True)
        lane = jax.lax.broadcasted_iota(jnp.int32, scores.shape, 1).astype(
            jnp.float32)
        j = jnp.min(jnp.where(scores >= conf, lane, 3.0e8), axis=1,
                    keepdims=True)
        x = xywh[:, 0:1]
        y = xywh[:, 1:2]
        w = xywh[:, 2:3]
        h = xywh[:, 3:4]
        x1 = x - w / 2.0
        y1 = y - h / 2.0
        x2 = x + w / 2.0
        y2 = y + h / 2.0
        valid = (obj > _CONF_T) & (conf > _CONF_T)
        sw = jnp.where(valid, conf, -1.0)
        zero = jnp.zeros_like(sw)
        out_ref[c:c + chunk, :] = jnp.concatenate(
            [x1, y1, x2, y2, conf, j, sw] + [zero] * (_DW - 7), axis=1)


def _nms_body(n_boxes, data_ref, order_ref, log_ref, out_ref, olog_ref,
              kx1_ref, ky1_ref, kx2_ref, ky2_ref, ka_ref):
    nc = log_ref.shape[-1]
    out_ref[...] = jnp.zeros((_OUT_R, _DW), jnp.float32)
    olog_ref[...] = jnp.zeros((_OUT_R, nc), jnp.float32)
    sentinel = jnp.full((_KS, _KL), -1.0e9, jnp.float32)
    kx1_ref[...] = sentinel
    ky1_ref[...] = sentinel
    kx2_ref[...] = sentinel
    ky2_ref[...] = sentinel
    ka_ref[...] = sentinel
    ids = (jax.lax.broadcasted_iota(jnp.int32, (_KS, _KL), 1) * _KS
           + jax.lax.broadcasted_iota(jnp.int32, (_KS, _KL), 0))
    lanew = jax.lax.broadcasted_iota(jnp.int32, (1, _DW), 1)

    def load(i):
        ii = jnp.minimum(i, n_boxes - 1)
        roww = order_ref[pl.ds(ii // _DW, 1), :]
        pos = jnp.sum(jnp.where(lanew == ii % _DW, roww, 0))
        row = data_ref[pl.ds(pos, 1), :]
        score = jnp.sum(jnp.where(lanew == 6, row, 0.0))
        return pos, row, score

    st0 = load(jnp.int32(0))

    def cond(st):
        i, cnt, _, _, score = st
        return (i < n_boxes) & (cnt < _MAX_DET) & (score > 0.0)

    def body(st):
        i, cnt, pos, row, _ = st

        def bc(k):
            return jnp.broadcast_to(row[0:1, k:k + 1], (_KS, _KL))

        off = bc(5) * _MAX_WH
        x1o = bc(0) + off
        y1o = bc(1) + off
        x2o = bc(2) + off
        y2o = bc(3) + off
        carea = (x2o - x1o) * (y2o - y1o)
        kx1 = kx1_ref[...]
        ky1 = ky1_ref[...]
        kx2 = kx2_ref[...]
        ky2 = ky2_ref[...]
        ka = ka_ref[...]
        ltx = jnp.maximum(kx1, x1o)
        lty = jnp.maximum(ky1, y1o)
        rbx = jnp.minimum(kx2, x2o)
        rby = jnp.minimum(ky2, y2o)
        inter = jnp.maximum(rbx - ltx, 0.0) * jnp.maximum(rby - lty, 0.0)
        iou = inter / (ka + carea - inter + 1e-9)
        hit = (ids < cnt) & (iou > _IOU_T)
        keep = jnp.max(jnp.where(hit, 1.0, 0.0)) == 0.0
        m = (ids == cnt) & keep
        kx1_ref[...] = jnp.where(m, x1o, kx1)
        ky1_ref[...] = jnp.where(m, y1o, ky1)
        kx2_ref[...] = jnp.where(m, x2o, kx2)
        ky2_ref[...] = jnp.where(m, y2o, ky2)
        ka_ref[...] = jnp.where(m, carea, ka)
        outrow = jnp.where(lanew == 7, pos.astype(jnp.float32), row)
        out_ref[pl.ds(cnt, 1), :] = jnp.where(keep, outrow, 0.0)
        logrow = log_ref[pl.ds(pos, 1), :]
        olog_ref[pl.ds(cnt, 1), :] = jnp.where(keep, logrow, 0.0)
        cnt2 = cnt + jnp.where(keep, 1, 0).astype(jnp.int32)
        pos2, row2, score2 = load(i + 1)
        return i + 1, cnt2, pos2, row2, score2

    jax.lax.while_loop(cond, body, (jnp.int32(0), jnp.int32(0)) + st0)


def kernel(prediction, logits):
    b, n, nf = prediction.shape
    nc = logits.shape[-1]
    f32 = jnp.float32

    data = pl.pallas_call(
        _prep_body,
        grid=(b,),
        in_specs=[pl.BlockSpec((None, n, nf), lambda bb: (bb, 0, 0))],
        out_specs=pl.BlockSpec((None, n, _DW), lambda bb: (bb, 0, 0)),
        out_shape=jax.ShapeDtypeStruct((b, n, _DW), f32),
    )(prediction)

    sw = data[..., 6]
    iot = jax.lax.broadcasted_iota(jnp.int32, (b, n), 1)
    _, order = jax.lax.sort((-sw, iot), dimension=1, is_stable=True,
                            num_keys=1)
    orderw = order.reshape(b, n // _DW, _DW)

    dets_full, logs_full = pl.pallas_call(
        functools.partial(_nms_body, n),
        grid=(b,),
        in_specs=[
            pl.BlockSpec((None, n, _DW), lambda bb: (bb, 0, 0)),
            pl.BlockSpec((None, n // _DW, _DW), lambda bb: (bb, 0, 0)),
            pl.BlockSpec((None, n, nc), lambda bb: (bb, 0, 0)),
        ],
        out_specs=[
            pl.BlockSpec((None, _OUT_R, _DW), lambda bb: (bb, 0, 0)),
            pl.BlockSpec((None, _OUT_R, nc), lambda bb: (bb, 0, 0)),
        ],
        out_shape=[
            jax.ShapeDtypeStruct((b, _OUT_R, _DW), f32),
            jax.ShapeDtypeStruct((b, _OUT_R, nc), f32),
        ],
        scratch_shapes=[pltpu.VMEM((_KS, _KL), f32) for _ in range(5)],
    )(data, orderw, logits)

    dets = dets_full[:, :_MAX_DET, :6]
    logs = logs_full[:, :_MAX_DET, :]
    return dets, logs
